# baseline (speedup 1.0000x reference)
import functools

import jax
import jax.numpy as jnp
from jax import lax
from jax.experimental import pallas as pl
from jax.experimental.pallas import tpu as pltpu
from jax.experimental.pallas import tpu_sc as plsc

EMB = 64
HID = 128
BATCH = 16384
NUSER = 1000000
NITEM = 100000
_WB = 2048
_UNQB = (NUSER // 2 + _WB - 1) // _WB
_INQB = (NITEM // 2 + _WB - 1) // _WB
_UTB = (NUSER + _WB - 1) // _WB
_ITB = (NITEM + _WB - 1) // _WB
_UHI = (_UTB - _UNQB) * _WB
_IHI = (_ITB - _INQB) * _WB

_NC, _NS = 2, 16
_NW = _NC * _NS
_BPW = BATCH // _NW


def _pairize_body(xlo_ref, xhi_ref, i_ref, o_ref):
    x = jnp.concatenate([xlo_ref[...], xhi_ref[...]], axis=0)
    o_ref[...] = lax.dot_general(
        x, i_ref[...], (((0,), (0,)), ((), ())),
        preferred_element_type=jnp.float32)


def _pairize(tabT, nblks, hi_shift_blks):
    eye = jnp.eye(2 * EMB, dtype=jnp.float32)
    return pl.pallas_call(
        _pairize_body,
        grid=(nblks,),
        in_specs=[
            pl.BlockSpec((EMB, _WB), lambda b: (0, b)),
            pl.BlockSpec((EMB, _WB), lambda b, s=hi_shift_blks: (0, b + s)),
            pl.BlockSpec((2 * EMB, 2 * EMB), lambda b: (0, 0)),
        ],
        out_specs=pl.BlockSpec((_WB, 128), lambda b: (b, 0)),
        out_shape=jax.ShapeDtypeStruct((nblks * _WB, 128), jnp.float32),
        compiler_params=pltpu.CompilerParams(
            fuse_transposed_lhs_in_matmul=True),
    )(tabT, tabT, eye)


def _sc_gather(uq_h, iq_h, upair, ipair, xup_hbm, xip_hbm,
               idx_v, rows_v, sem):
    wid = lax.axis_index("s") * _NC + lax.axis_index("c")
    base = wid * _BPW
    pltpu.sync_copy(uq_h.at[pl.ds(base, _BPW)], idx_v)
    pltpu.async_copy(upair.at[idx_v], rows_v, sem).wait()
    pltpu.sync_copy(rows_v, xup_hbm.at[pl.ds(base, _BPW)])
    pltpu.sync_copy(iq_h.at[pl.ds(base, _BPW)], idx_v)
    pltpu.async_copy(ipair.at[idx_v], rows_v, sem).wait()
    pltpu.sync_copy(rows_v, xip_hbm.at[pl.ds(base, _BPW)])


def _mlp_body(xu_ref, xi_ref, up_ref, ip_ref, w1a_ref, w1b_ref, b1_ref,
              w2r_ref, b2_ref, o_ref):
    u = jnp.where(up_ref[...] > 0, xu_ref[:, EMB:], xu_ref[:, :EMB])
    v = jnp.where(ip_ref[...] > 0, xi_ref[:, EMB:], xi_ref[:, :EMB])
    h = jnp.dot(u, w1a_ref[...], preferred_element_type=jnp.float32)
    h += jnp.dot(v, w1b_ref[...], preferred_element_type=jnp.float32)
    h = jnp.maximum(h + b1_ref[...], 0.0)
    o_ref[...] = jnp.sum(h * w2r_ref[...], axis=1) + b2_ref[0, 0]


@jax.jit
def kernel(user, item, user_emb, item_emb, W1, b1, W2, b2):
    user = user.astype(jnp.int32)
    item = item.astype(jnp.int32)

    upair = _pairize(user_emb.T, _UNQB, _UTB - _UNQB)
    ipair = _pairize(item_emb.T, _INQB, _ITB - _INQB)

    uq = jnp.where(user < _UHI, user, user - _UHI)
    iq = jnp.where(item < _IHI, item, item - _IHI)
    up = (user >= _UHI).astype(jnp.int32).reshape(BATCH, 1)
    ip = (item >= _IHI).astype(jnp.int32).reshape(BATCH, 1)

    gather = functools.partial(
        pl.kernel,
        mesh=plsc.VectorSubcoreMesh(core_axis_name="c", subcore_axis_name="s"),
        out_type=[
            jax.ShapeDtypeStruct((BATCH, 128), jnp.float32),
            jax.ShapeDtypeStruct((BATCH, 128), jnp.float32),
        ],
        scratch_types=[
            pltpu.VMEM((_BPW,), jnp.int32),
            pltpu.VMEM((_BPW, 128), jnp.float32),
            pltpu.SemaphoreType.DMA,
        ],
        compiler_params=pltpu.CompilerParams(
            use_tc_tiling_on_sc=False, needs_layout_passes=False),
    )(_sc_gather)
    xup, xip = gather(uq, iq, upair, ipair)

    bm = 2048
    w1a = W1[:EMB]
    w1b = W1[EMB:]
    b1r = b1.reshape(1, HID)
    w2r = W2.reshape(1, HID)
    b2r = b2.reshape(1, 1)
    out = pl.pallas_call(
        _mlp_body,
        grid=(BATCH // bm,),
        in_specs=[
            pl.BlockSpec((bm, 128), lambda i: (i, 0)),
            pl.BlockSpec((bm, 128), lambda i: (i, 0)),
            pl.BlockSpec((bm, 1), lambda i: (i, 0)),
            pl.BlockSpec((bm, 1), lambda i: (i, 0)),
            pl.BlockSpec((EMB, HID), lambda i: (0, 0)),
            pl.BlockSpec((EMB, HID), lambda i: (0, 0)),
            pl.BlockSpec((1, HID), lambda i: (0, 0)),
            pl.BlockSpec((1, HID), lambda i: (0, 0)),
            pl.BlockSpec((1, 1), lambda i: (0, 0)),
        ],
        out_specs=pl.BlockSpec((bm,), lambda i: (i,)),
        out_shape=jax.ShapeDtypeStruct((BATCH,), jnp.float32),
    )(xup, xip, up, ip, w1a, w1b, b1r, w2r, b2r)
    return out


# speedup vs baseline: 1.3082x; 1.3082x over previous
import functools

import jax
import jax.numpy as jnp
from jax import lax
from jax.experimental import pallas as pl
from jax.experimental.pallas import tpu as pltpu
from jax.experimental.pallas import tpu_sc as plsc

EMB = 64
HID = 128
BATCH = 16384
NUSER = 1000000
NITEM = 100000
_WB = 4096
_UNQB = (NUSER // 2 + _WB - 1) // _WB
_INQB = (NITEM // 2 + _WB - 1) // _WB
_UTB = (NUSER + _WB - 1) // _WB
_ITB = (NITEM + _WB - 1) // _WB
_UHI = (_UTB - _UNQB) * _WB
_IHI = (_ITB - _INQB) * _WB

_NC, _NS = 2, 16
_NW = _NC * _NS
_BPW = BATCH // _NW


def _pairize_body(xlo_ref, xhi_ref, i_ref, o_ref):
    x = jnp.concatenate([xlo_ref[...], xhi_ref[...]], axis=0)
    o_ref[...] = lax.dot_general(
        x, i_ref[...], (((0,), (0,)), ((), ())),
        preferred_element_type=jnp.float32)


def _pairize(tabT, nblks, hi_shift_blks):
    eye = jnp.eye(2 * EMB, dtype=jnp.float32)
    return pl.pallas_call(
        _pairize_body,
        grid=(nblks,),
        in_specs=[
            pl.BlockSpec((EMB, _WB), lambda b: (0, b)),
            pl.BlockSpec((EMB, _WB), lambda b, s=hi_shift_blks: (0, b + s)),
            pl.BlockSpec((2 * EMB, 2 * EMB), lambda b: (0, 0)),
        ],
        out_specs=pl.BlockSpec((_WB, 128), lambda b: (b, 0)),
        out_shape=jax.ShapeDtypeStruct((nblks * _WB, 128), jnp.float32),
        compiler_params=pltpu.CompilerParams(
            fuse_transposed_lhs_in_matmul=True),
    )(tabT, tabT, eye)


def _sc_gather(uq_h, iq_h, upair, ipair, xup_hbm, xip_hbm,
               idx_v, rows_v, sem):
    wid = lax.axis_index("s") * _NC + lax.axis_index("c")
    base = wid * _BPW
    pltpu.sync_copy(uq_h.at[pl.ds(base, _BPW)], idx_v)
    pltpu.async_copy(upair.at[idx_v], rows_v, sem).wait()
    pltpu.sync_copy(rows_v, xup_hbm.at[pl.ds(base, _BPW)])
    pltpu.sync_copy(iq_h.at[pl.ds(base, _BPW)], idx_v)
    pltpu.async_copy(ipair.at[idx_v], rows_v, sem).wait()
    pltpu.sync_copy(rows_v, xip_hbm.at[pl.ds(base, _BPW)])


def _mlp_body(xu_ref, xi_ref, up_ref, ip_ref, w1a_ref, w1b_ref, b1_ref,
              w2r_ref, b2_ref, o_ref):
    u = jnp.where(up_ref[...] > 0, xu_ref[:, EMB:], xu_ref[:, :EMB])
    v = jnp.where(ip_ref[...] > 0, xi_ref[:, EMB:], xi_ref[:, :EMB])
    h = jnp.dot(u, w1a_ref[...], preferred_element_type=jnp.float32)
    h += jnp.dot(v, w1b_ref[...], preferred_element_type=jnp.float32)
    h = jnp.maximum(h + b1_ref[...], 0.0)
    o_ref[...] = jnp.sum(h * w2r_ref[...], axis=1) + b2_ref[0, 0]


@jax.jit
def kernel(user, item, user_emb, item_emb, W1, b1, W2, b2):
    user = user.astype(jnp.int32)
    item = item.astype(jnp.int32)

    upair = _pairize(user_emb.T, _UNQB, _UTB - _UNQB)
    ipair = _pairize(item_emb.T, _INQB, _ITB - _INQB)

    uq = jnp.where(user < _UHI, user, user - _UHI)
    iq = jnp.where(item < _IHI, item, item - _IHI)
    up = (user >= _UHI).astype(jnp.int32).reshape(BATCH, 1)
    ip = (item >= _IHI).astype(jnp.int32).reshape(BATCH, 1)

    gather = functools.partial(
        pl.kernel,
        mesh=plsc.VectorSubcoreMesh(core_axis_name="c", subcore_axis_name="s"),
        out_type=[
            jax.ShapeDtypeStruct((BATCH, 128), jnp.float32),
            jax.ShapeDtypeStruct((BATCH, 128), jnp.float32),
        ],
        scratch_types=[
            pltpu.VMEM((_BPW,), jnp.int32),
            pltpu.VMEM((_BPW, 128), jnp.float32),
            pltpu.SemaphoreType.DMA,
        ],
        compiler_params=pltpu.CompilerParams(
            use_tc_tiling_on_sc=False, needs_layout_passes=False),
    )(_sc_gather)
    xup, xip = gather(uq, iq, upair, ipair)

    bm = 2048
    w1a = W1[:EMB]
    w1b = W1[EMB:]
    b1r = b1.reshape(1, HID)
    w2r = W2.reshape(1, HID)
    b2r = b2.reshape(1, 1)
    out = pl.pallas_call(
        _mlp_body,
        grid=(BATCH // bm,),
        in_specs=[
            pl.BlockSpec((bm, 128), lambda i: (i, 0)),
            pl.BlockSpec((bm, 128), lambda i: (i, 0)),
            pl.BlockSpec((bm, 1), lambda i: (i, 0)),
            pl.BlockSpec((bm, 1), lambda i: (i, 0)),
            pl.BlockSpec((EMB, HID), lambda i: (0, 0)),
            pl.BlockSpec((EMB, HID), lambda i: (0, 0)),
            pl.BlockSpec((1, HID), lambda i: (0, 0)),
            pl.BlockSpec((1, HID), lambda i: (0, 0)),
            pl.BlockSpec((1, 1), lambda i: (0, 0)),
        ],
        out_specs=pl.BlockSpec((bm,), lambda i: (i,)),
        out_shape=jax.ShapeDtypeStruct((BATCH,), jnp.float32),
    )(xup, xip, up, ip, w1a, w1b, b1r, w2r, b2r)
    return out


# speedup vs baseline: 1.4654x; 1.1202x over previous
import functools

import jax
import jax.numpy as jnp
from jax import lax
from jax.experimental import pallas as pl
from jax.experimental.pallas import tpu as pltpu
from jax.experimental.pallas import tpu_sc as plsc

EMB = 64
HID = 128
BATCH = 16384
NUSER = 1000000
NITEM = 100000
_WB = 8192
_UNQB = (NUSER // 2 + _WB - 1) // _WB
_INQB = (NITEM // 2 + _WB - 1) // _WB
_UTB = (NUSER + _WB - 1) // _WB
_ITB = (NITEM + _WB - 1) // _WB
_UHI = (_UTB - _UNQB) * _WB
_IHI = (_ITB - _INQB) * _WB

_NC, _NS = 2, 16
_NW = _NC * _NS
_BPW = BATCH // _NW


def _pairize_body(xlo_ref, xhi_ref, i_ref, o_ref):
    x = jnp.concatenate([xlo_ref[...], xhi_ref[...]], axis=0)
    o_ref[...] = lax.dot_general(
        x, i_ref[...], (((0,), (0,)), ((), ())),
        preferred_element_type=jnp.float32)


def _pairize(tabT, nblks, hi_shift_blks):
    eye = jnp.eye(2 * EMB, dtype=jnp.float32)
    return pl.pallas_call(
        _pairize_body,
        grid=(nblks,),
        in_specs=[
            pl.BlockSpec((EMB, _WB), lambda b: (0, b)),
            pl.BlockSpec((EMB, _WB), lambda b, s=hi_shift_blks: (0, b + s)),
            pl.BlockSpec((2 * EMB, 2 * EMB), lambda b: (0, 0)),
        ],
        out_specs=pl.BlockSpec((_WB, 128), lambda b: (b, 0)),
        out_shape=jax.ShapeDtypeStruct((nblks * _WB, 128), jnp.float32),
        compiler_params=pltpu.CompilerParams(
            fuse_transposed_lhs_in_matmul=True),
    )(tabT, tabT, eye)


def _sc_gather(uq_h, iq_h, upair, ipair, xup_hbm, xip_hbm,
               idx_v, rows_v, sem):
    wid = lax.axis_index("s") * _NC + lax.axis_index("c")
    base = wid * _BPW
    pltpu.sync_copy(uq_h.at[pl.ds(base, _BPW)], idx_v)
    pltpu.async_copy(upair.at[idx_v], rows_v, sem).wait()
    pltpu.sync_copy(rows_v, xup_hbm.at[pl.ds(base, _BPW)])
    pltpu.sync_copy(iq_h.at[pl.ds(base, _BPW)], idx_v)
    pltpu.async_copy(ipair.at[idx_v], rows_v, sem).wait()
    pltpu.sync_copy(rows_v, xip_hbm.at[pl.ds(base, _BPW)])


def _mlp_body(xu_ref, xi_ref, up_ref, ip_ref, w1a_ref, w1b_ref, b1_ref,
              w2r_ref, b2_ref, o_ref):
    u = jnp.where(up_ref[...] > 0, xu_ref[:, EMB:], xu_ref[:, :EMB])
    v = jnp.where(ip_ref[...] > 0, xi_ref[:, EMB:], xi_ref[:, :EMB])
    h = jnp.dot(u, w1a_ref[...], preferred_element_type=jnp.float32)
    h += jnp.dot(v, w1b_ref[...], preferred_element_type=jnp.float32)
    h = jnp.maximum(h + b1_ref[...], 0.0)
    o_ref[...] = jnp.sum(h * w2r_ref[...], axis=1) + b2_ref[0, 0]


@jax.jit
def kernel(user, item, user_emb, item_emb, W1, b1, W2, b2):
    user = user.astype(jnp.int32)
    item = item.astype(jnp.int32)

    upair = _pairize(user_emb.T, _UNQB, _UTB - _UNQB)
    ipair = _pairize(item_emb.T, _INQB, _ITB - _INQB)

    uq = jnp.where(user < _UHI, user, user - _UHI)
    iq = jnp.where(item < _IHI, item, item - _IHI)
    up = (user >= _UHI).astype(jnp.int32).reshape(BATCH, 1)
    ip = (item >= _IHI).astype(jnp.int32).reshape(BATCH, 1)

    gather = functools.partial(
        pl.kernel,
        mesh=plsc.VectorSubcoreMesh(core_axis_name="c", subcore_axis_name="s"),
        out_type=[
            jax.ShapeDtypeStruct((BATCH, 128), jnp.float32),
            jax.ShapeDtypeStruct((BATCH, 128), jnp.float32),
        ],
        scratch_types=[
            pltpu.VMEM((_BPW,), jnp.int32),
            pltpu.VMEM((_BPW, 128), jnp.float32),
            pltpu.SemaphoreType.DMA,
        ],
        compiler_params=pltpu.CompilerParams(
            use_tc_tiling_on_sc=False, needs_layout_passes=False),
    )(_sc_gather)
    xup, xip = gather(uq, iq, upair, ipair)

    bm = 2048
    w1a = W1[:EMB]
    w1b = W1[EMB:]
    b1r = b1.reshape(1, HID)
    w2r = W2.reshape(1, HID)
    b2r = b2.reshape(1, 1)
    out = pl.pallas_call(
        _mlp_body,
        grid=(BATCH // bm,),
        in_specs=[
            pl.BlockSpec((bm, 128), lambda i: (i, 0)),
            pl.BlockSpec((bm, 128), lambda i: (i, 0)),
            pl.BlockSpec((bm, 1), lambda i: (i, 0)),
            pl.BlockSpec((bm, 1), lambda i: (i, 0)),
            pl.BlockSpec((EMB, HID), lambda i: (0, 0)),
            pl.BlockSpec((EMB, HID), lambda i: (0, 0)),
            pl.BlockSpec((1, HID), lambda i: (0, 0)),
            pl.BlockSpec((1, HID), lambda i: (0, 0)),
            pl.BlockSpec((1, 1), lambda i: (0, 0)),
        ],
        out_specs=pl.BlockSpec((bm,), lambda i: (i,)),
        out_shape=jax.ShapeDtypeStruct((BATCH,), jnp.float32),
    )(xup, xip, up, ip, w1a, w1b, b1r, w2r, b2r)
    return out


# speedup vs baseline: 1.4979x; 1.0221x over previous
import functools

import jax
import jax.numpy as jnp
from jax import lax
from jax.experimental import pallas as pl
from jax.experimental.pallas import tpu as pltpu
from jax.experimental.pallas import tpu_sc as plsc

EMB = 64
HID = 128
BATCH = 16384
NUSER = 1000000
NITEM = 100000
_WB = 16384
_UNQB = (NUSER // 2 + _WB - 1) // _WB
_INQB = (NITEM // 2 + _WB - 1) // _WB
_UTB = (NUSER + _WB - 1) // _WB
_ITB = (NITEM + _WB - 1) // _WB
_UHI = (_UTB - _UNQB) * _WB
_IHI = (_ITB - _INQB) * _WB

_NC, _NS = 2, 16
_NW = _NC * _NS
_BPW = BATCH // _NW


def _pairize_body(xlo_ref, xhi_ref, i_ref, o_ref):
    x = jnp.concatenate([xlo_ref[...], xhi_ref[...]], axis=0)
    o_ref[...] = lax.dot_general(
        x, i_ref[...], (((0,), (0,)), ((), ())),
        preferred_element_type=jnp.float32)


def _pairize(tabT, nblks, hi_shift_blks):
    eye = jnp.eye(2 * EMB, dtype=jnp.float32)
    return pl.pallas_call(
        _pairize_body,
        grid=(nblks,),
        in_specs=[
            pl.BlockSpec((EMB, _WB), lambda b: (0, b)),
            pl.BlockSpec((EMB, _WB), lambda b, s=hi_shift_blks: (0, b + s)),
            pl.BlockSpec((2 * EMB, 2 * EMB), lambda b: (0, 0)),
        ],
        out_specs=pl.BlockSpec((_WB, 128), lambda b: (b, 0)),
        out_shape=jax.ShapeDtypeStruct((nblks * _WB, 128), jnp.float32),
        compiler_params=pltpu.CompilerParams(
            fuse_transposed_lhs_in_matmul=True),
    )(tabT, tabT, eye)


def _sc_gather(uq_h, iq_h, upair, ipair, xup_hbm, xip_hbm,
               idx_v, rows_v, sem):
    wid = lax.axis_index("s") * _NC + lax.axis_index("c")
    base = wid * _BPW
    pltpu.sync_copy(uq_h.at[pl.ds(base, _BPW)], idx_v)
    pltpu.async_copy(upair.at[idx_v], rows_v, sem).wait()
    pltpu.sync_copy(rows_v, xup_hbm.at[pl.ds(base, _BPW)])
    pltpu.sync_copy(iq_h.at[pl.ds(base, _BPW)], idx_v)
    pltpu.async_copy(ipair.at[idx_v], rows_v, sem).wait()
    pltpu.sync_copy(rows_v, xip_hbm.at[pl.ds(base, _BPW)])


def _mlp_body(xu_ref, xi_ref, up_ref, ip_ref, w1a_ref, w1b_ref, b1_ref,
              w2r_ref, b2_ref, o_ref):
    u = jnp.where(up_ref[...] > 0, xu_ref[:, EMB:], xu_ref[:, :EMB])
    v = jnp.where(ip_ref[...] > 0, xi_ref[:, EMB:], xi_ref[:, :EMB])
    h = jnp.dot(u, w1a_ref[...], preferred_element_type=jnp.float32)
    h += jnp.dot(v, w1b_ref[...], preferred_element_type=jnp.float32)
    h = jnp.maximum(h + b1_ref[...], 0.0)
    o_ref[...] = jnp.sum(h * w2r_ref[...], axis=1) + b2_ref[0, 0]


@jax.jit
def kernel(user, item, user_emb, item_emb, W1, b1, W2, b2):
    user = user.astype(jnp.int32)
    item = item.astype(jnp.int32)

    upair = _pairize(user_emb.T, _UNQB, _UTB - _UNQB)
    ipair = _pairize(item_emb.T, _INQB, _ITB - _INQB)

    uq = jnp.where(user < _UHI, user, user - _UHI)
    iq = jnp.where(item < _IHI, item, item - _IHI)
    up = (user >= _UHI).astype(jnp.int32).reshape(BATCH, 1)
    ip = (item >= _IHI).astype(jnp.int32).reshape(BATCH, 1)

    gather = functools.partial(
        pl.kernel,
        mesh=plsc.VectorSubcoreMesh(core_axis_name="c", subcore_axis_name="s"),
        out_type=[
            jax.ShapeDtypeStruct((BATCH, 128), jnp.float32),
            jax.ShapeDtypeStruct((BATCH, 128), jnp.float32),
        ],
        scratch_types=[
            pltpu.VMEM((_BPW,), jnp.int32),
            pltpu.VMEM((_BPW, 128), jnp.float32),
            pltpu.SemaphoreType.DMA,
        ],
        compiler_params=pltpu.CompilerParams(
            use_tc_tiling_on_sc=False, needs_layout_passes=False),
    )(_sc_gather)
    xup, xip = gather(uq, iq, upair, ipair)

    bm = 2048
    w1a = W1[:EMB]
    w1b = W1[EMB:]
    b1r = b1.reshape(1, HID)
    w2r = W2.reshape(1, HID)
    b2r = b2.reshape(1, 1)
    out = pl.pallas_call(
        _mlp_body,
        grid=(BATCH // bm,),
        in_specs=[
            pl.BlockSpec((bm, 128), lambda i: (i, 0)),
            pl.BlockSpec((bm, 128), lambda i: (i, 0)),
            pl.BlockSpec((bm, 1), lambda i: (i, 0)),
            pl.BlockSpec((bm, 1), lambda i: (i, 0)),
            pl.BlockSpec((EMB, HID), lambda i: (0, 0)),
            pl.BlockSpec((EMB, HID), lambda i: (0, 0)),
            pl.BlockSpec((1, HID), lambda i: (0, 0)),
            pl.BlockSpec((1, HID), lambda i: (0, 0)),
            pl.BlockSpec((1, 1), lambda i: (0, 0)),
        ],
        out_specs=pl.BlockSpec((bm,), lambda i: (i,)),
        out_shape=jax.ShapeDtypeStruct((BATCH,), jnp.float32),
    )(xup, xip, up, ip, w1a, w1b, b1r, w2r, b2r)
    return out
